# baseline (device time: 38725 ns/iter reference)
import jax
import jax.numpy as jnp
from jax import lax
from jax.experimental import pallas as pl
from jax.experimental.pallas import tpu as pltpu

N_DEV = 4
SEQ = 1024
HALO = 128
KV = SEQ + 2 * HALO
HQ = 8
DH = 128
QB = 256
WIN = QB + 2 * HALO
SCALE = 0.08838834764831843


def kernel(x, Wq, K_ext, V_ext, Wo):
    x2 = x.reshape(SEQ, HQ * DH)
    K2 = K_ext.reshape(SEQ, HQ * DH)
    V2 = V_ext.reshape(SEQ, HQ * DH)

    def body(x_ref, wq_ref, k_ref, v_ref, wo_ref, out_ref,
             kv_k, kv_v, q_scr, ctx_scr, send_sems, recv_sems):
        my = lax.axis_index("i")
        left = (my - 1) % N_DEV
        right = (my + 1) % N_DEV

        barrier_sem = pltpu.get_barrier_semaphore()
        for nbr in (left, right):
            pl.semaphore_signal(
                barrier_sem, inc=1,
                device_id=(nbr,), device_id_type=pl.DeviceIdType.MESH,
            )
        pl.semaphore_wait(barrier_sem, 2)

        kv_k[pl.ds(HALO, SEQ), :] = k_ref[...].astype(jnp.bfloat16)
        kv_v[pl.ds(HALO, SEQ), :] = v_ref[...].astype(jnp.bfloat16)

        descs = []
        for idx, (buf, src_off, dst_off, tgt) in enumerate([
            (kv_k, HALO, SEQ + HALO, left),
            (kv_v, HALO, SEQ + HALO, left),
            (kv_k, SEQ, 0, right),
            (kv_v, SEQ, 0, right),
        ]):
            d = pltpu.make_async_remote_copy(
                src_ref=buf.at[pl.ds(src_off, HALO), :],
                dst_ref=buf.at[pl.ds(dst_off, HALO), :],
                send_sem=send_sems.at[idx],
                recv_sem=recv_sems.at[idx],
                device_id=(tgt,),
                device_id_type=pl.DeviceIdType.MESH,
            )
            d.start()
            descs.append(d)

        q_scr[...] = jnp.dot(
            x_ref[...].astype(jnp.bfloat16),
            wq_ref[...].astype(jnp.bfloat16),
            preferred_element_type=jnp.float32,
        ).astype(jnp.bfloat16)

        for d in descs:
            d.wait()

        lo = jnp.where(my == 0, HALO, 0)
        hi = jnp.where(my == N_DEV - 1, SEQ + HALO, KV)

        for h in range(HQ):
            hs = slice(h * DH, (h + 1) * DH)
            for b in range(SEQ // QB):
                q = q_scr[pl.ds(b * QB, QB), hs]
                k = kv_k[pl.ds(b * QB, WIN), hs]
                s = lax.dot_general(
                    q, k, (((1,), (1,)), ((), ())),
                    preferred_element_type=jnp.float32,
                ) * SCALE
                ii = lax.broadcasted_iota(jnp.int32, (QB, WIN), 0)
                jj = lax.broadcasted_iota(jnp.int32, (QB, WIN), 1)
                diff = jj - ii
                jbuf = jj + b * QB
                m = (diff >= 0) & (diff <= 2 * HALO) & (jbuf >= lo) & (jbuf < hi)
                s = jnp.where(m, s, -1e9)
                mx = jnp.max(s, axis=1, keepdims=True)
                w = jnp.exp(s - mx)
                p = (w / jnp.sum(w, axis=1, keepdims=True)).astype(jnp.bfloat16)
                v = kv_v[pl.ds(b * QB, WIN), hs]
                ctx = jnp.dot(p, v, preferred_element_type=jnp.float32)
                ctx_scr[pl.ds(b * QB, QB), hs] = ctx.astype(jnp.bfloat16)

        out_ref[...] = jnp.dot(
            ctx_scr[...],
            wo_ref[...].astype(jnp.bfloat16),
            preferred_element_type=jnp.float32,
        )

    out = pl.pallas_call(
        body,
        out_shape=jax.ShapeDtypeStruct((SEQ, HQ * DH), jnp.float32),
        in_specs=[pl.BlockSpec(memory_space=pltpu.VMEM)] * 5,
        out_specs=pl.BlockSpec(memory_space=pltpu.VMEM),
        scratch_shapes=[
            pltpu.VMEM((KV, HQ * DH), jnp.bfloat16),
            pltpu.VMEM((KV, HQ * DH), jnp.bfloat16),
            pltpu.VMEM((SEQ, HQ * DH), jnp.bfloat16),
            pltpu.VMEM((SEQ, HQ * DH), jnp.bfloat16),
            pltpu.SemaphoreType.DMA((4,)),
            pltpu.SemaphoreType.DMA((4,)),
        ],
        compiler_params=pltpu.CompilerParams(collective_id=0),
    )(x2, Wq, K2, V2, Wo)
    return out.reshape(1, SEQ, HQ * DH)


# device time: 29545 ns/iter; 1.3107x vs baseline; 1.3107x over previous
import jax
import jax.numpy as jnp
from jax import lax
from jax.experimental import pallas as pl
from jax.experimental.pallas import tpu as pltpu

N_DEV = 4
SEQ = 1024
HALO = 128
KV = SEQ + 2 * HALO
HQ = 8
DH = 128
QB = 256
WIN = QB + 2 * HALO
SCALE = 0.08838834764831843


def kernel(x, Wq, K_ext, V_ext, Wo):
    x2 = x.reshape(SEQ, HQ * DH)
    K2 = K_ext.reshape(SEQ, HQ * DH)
    V2 = V_ext.reshape(SEQ, HQ * DH)

    def body(x_ref, wq_ref, k_ref, v_ref, wo_ref, out_ref,
             kv_k, kv_v, q_scr, ctx_scr, send_sems, recv_sems):
        my = lax.axis_index("i")
        left = (my - 1) % N_DEV
        right = (my + 1) % N_DEV

        barrier_sem = pltpu.get_barrier_semaphore()
        for nbr in (left, right):
            pl.semaphore_signal(
                barrier_sem, inc=1,
                device_id=(nbr,), device_id_type=pl.DeviceIdType.MESH,
            )
        pl.semaphore_wait(barrier_sem, 2)

        kv_k[pl.ds(HALO, HALO), :] = k_ref[pl.ds(0, HALO), :].astype(jnp.bfloat16)
        kv_v[pl.ds(HALO, HALO), :] = v_ref[pl.ds(0, HALO), :].astype(jnp.bfloat16)
        kv_k[pl.ds(SEQ, HALO), :] = k_ref[pl.ds(SEQ - HALO, HALO), :].astype(jnp.bfloat16)
        kv_v[pl.ds(SEQ, HALO), :] = v_ref[pl.ds(SEQ - HALO, HALO), :].astype(jnp.bfloat16)

        descs = []
        for idx, (buf, src_off, dst_off, tgt) in enumerate([
            (kv_k, HALO, SEQ + HALO, left),
            (kv_v, HALO, SEQ + HALO, left),
            (kv_k, SEQ, 0, right),
            (kv_v, SEQ, 0, right),
        ]):
            d = pltpu.make_async_remote_copy(
                src_ref=buf.at[pl.ds(src_off, HALO), :],
                dst_ref=buf.at[pl.ds(dst_off, HALO), :],
                send_sem=send_sems.at[idx],
                recv_sem=recv_sems.at[idx],
                device_id=(tgt,),
                device_id_type=pl.DeviceIdType.MESH,
            )
            d.start()
            descs.append(d)

        mid = SEQ - 2 * HALO
        kv_k[pl.ds(2 * HALO, mid), :] = k_ref[pl.ds(HALO, mid), :].astype(jnp.bfloat16)
        kv_v[pl.ds(2 * HALO, mid), :] = v_ref[pl.ds(HALO, mid), :].astype(jnp.bfloat16)
        q_scr[...] = (
            jnp.dot(
                x_ref[...].astype(jnp.bfloat16),
                wq_ref[...].astype(jnp.bfloat16),
                preferred_element_type=jnp.float32,
            )
            * SCALE
        ).astype(jnp.bfloat16)

        lo = jnp.where(my == 0, HALO, 0)
        hi = jnp.where(my == N_DEV - 1, SEQ + HALO, KV)

        def attend_block(b):
            ii = lax.broadcasted_iota(jnp.int32, (QB, WIN), 0)
            jj = lax.broadcasted_iota(jnp.int32, (QB, WIN), 1)
            diff = jj - ii
            jbuf = jj + b * QB
            m = (diff >= 0) & (diff <= 2 * HALO) & (jbuf >= lo) & (jbuf < hi)
            for h in range(HQ):
                hs = slice(h * DH, (h + 1) * DH)
                q = q_scr[pl.ds(b * QB, QB), hs]
                k = kv_k[pl.ds(b * QB, WIN), hs]
                s = lax.dot_general(
                    q, k, (((1,), (1,)), ((), ())),
                    preferred_element_type=jnp.float32,
                )
                w = jnp.where(m, jnp.exp(s), 0.0)
                p = (w / jnp.sum(w, axis=1, keepdims=True)).astype(jnp.bfloat16)
                v = kv_v[pl.ds(b * QB, WIN), hs]
                ctx = jnp.dot(p, v, preferred_element_type=jnp.float32)
                ctx_scr[pl.ds(b * QB, QB), hs] = ctx.astype(jnp.bfloat16)

        attend_block(1)
        attend_block(2)
        descs[2].wait_recv()
        descs[3].wait_recv()
        attend_block(0)
        descs[0].wait_recv()
        descs[1].wait_recv()
        attend_block(3)
        for d in descs:
            d.wait_send()

        out_ref[...] = jnp.dot(
            ctx_scr[...],
            wo_ref[...].astype(jnp.bfloat16),
            preferred_element_type=jnp.float32,
        )

    out = pl.pallas_call(
        body,
        out_shape=jax.ShapeDtypeStruct((SEQ, HQ * DH), jnp.float32),
        in_specs=[pl.BlockSpec(memory_space=pltpu.VMEM)] * 5,
        out_specs=pl.BlockSpec(memory_space=pltpu.VMEM),
        scratch_shapes=[
            pltpu.VMEM((KV, HQ * DH), jnp.bfloat16),
            pltpu.VMEM((KV, HQ * DH), jnp.bfloat16),
            pltpu.VMEM((SEQ, HQ * DH), jnp.bfloat16),
            pltpu.VMEM((SEQ, HQ * DH), jnp.bfloat16),
            pltpu.SemaphoreType.DMA((4,)),
            pltpu.SemaphoreType.DMA((4,)),
        ],
        compiler_params=pltpu.CompilerParams(collective_id=0),
    )(x2, Wq, K2, V2, Wo)
    return out.reshape(1, SEQ, HQ * DH)
